# baseline (speedup 1.0000x reference)
import functools

import jax
import jax.numpy as jnp
from jax import lax
from jax.experimental import pallas as pl
from jax.experimental.pallas import tpu as pltpu
from jax.experimental.pallas import tpu_sc as plsc

N = 10000
E = 320000
F = 128
H = 32
G = 64
CLS = 10

NC = 2
NS = 16
NW = NC * NS
CH = 128
K = 80
EPAD = NW * K * CH
NPAD = 10240
RP = NPAD // 4
RN = N // 4
RS = RP // NS
D = 8
NSLOT = 2 * D
KP = 5
EPOOL = NW * KP * CH


def _segsum_body(k, p_hbm, src_hbm, dst_hbm, zeros_hbm, out_hbm, srcb, dstb,
                 rows, acc, pshr, semg, sems):
    c = lax.axis_index("c")
    s = lax.axis_index("s")
    wid = c * NS + s
    rz = NPAD // NS
    pltpu.sync_copy(zeros_hbm.at[pl.ds(s * rz, rz)], acc.at[pl.ds(s * rz, rz)])
    pltpu.sync_copy(p_hbm.at[pl.ds(s * rz, rz)], pshr.at[pl.ds(s * rz, rz)])
    pltpu.sync_copy(src_hbm.at[pl.ds(wid * k, k)], srcb)
    pltpu.sync_copy(dst_hbm.at[pl.ds(wid * k, k)], dstb)
    plsc.subcore_barrier()

    for b in range(min(D, k)):
        pltpu.async_copy(pshr.at[srcb.at[b]], rows.at[b], semg)

    def step(j, carry):
        jm = lax.rem(j, NSLOT)
        pltpu.make_async_copy(pshr.at[srcb.at[j]], rows.at[jm], semg).wait()
        pltpu.async_copy(rows.at[jm], acc.at[dstb.at[j]], sems, add=True)

        @pl.when(j >= D)
        def _():
            jd = j - D
            pltpu.make_async_copy(rows.at[lax.rem(jd, NSLOT)],
                                  acc.at[dstb.at[jd]], sems).wait()

        @pl.when(j + D < k)
        def _():
            pltpu.async_copy(pshr.at[srcb.at[j + D]],
                             rows.at[lax.rem(j + D, NSLOT)], semg)

        return carry

    lax.fori_loop(0, k, step, 0)

    def drain(j, carry):
        pltpu.make_async_copy(rows.at[lax.rem(j, NSLOT)],
                              acc.at[dstb.at[j]], sems).wait()
        return carry

    lax.fori_loop(max(k - D, 0), k, drain, 0)
    plsc.subcore_barrier()
    pltpu.sync_copy(acc.at[pl.ds(s * rz, rz)], out_hbm.at[c, pl.ds(s * rz, rz)])


@functools.lru_cache(maxsize=4)
def _make_segsum(k):
    return pl.kernel(
        functools.partial(_segsum_body, k),
        out_type=jax.ShapeDtypeStruct((NC, NPAD, H), jnp.float32),
        mesh=plsc.VectorSubcoreMesh(core_axis_name="c", subcore_axis_name="s"),
        scratch_types=[
            pltpu.VMEM((k, CH), jnp.int32),
            pltpu.VMEM((k, CH), jnp.int32),
            pltpu.VMEM((NSLOT, CH, H), jnp.float32),
            pltpu.VMEM_SHARED((NPAD, H), jnp.float32),
            pltpu.VMEM_SHARED((NPAD, H), jnp.float32),
            pltpu.SemaphoreType.DMA,
            pltpu.SemaphoreType.DMA,
        ],
        compiler_params=pltpu.CompilerParams(use_tc_tiling_on_sc=False),
    )


def _fold4(v, n):
    ri = lax.rem(lax.broadcasted_iota(jnp.int32, (F, F), 0), H)
    cj = lax.rem(lax.broadcasted_iota(jnp.int32, (F, F), 1), H)
    m = (ri == cj).astype(jnp.float32)
    return jnp.dot(v, m, preferred_element_type=jnp.float32) / n


def _mlp_bn(p, pa, pb, b1, w2big, b2, gam, bet):
    z = jnp.maximum(p + pa + pb + b1, 0.0)
    z = jnp.maximum(jnp.dot(z, w2big, preferred_element_type=jnp.float32) + b2, 0.0)
    zs = z[0:RN]
    mu = _fold4(jnp.sum(zs, axis=0, keepdims=True), float(N))
    zc = z - mu
    zcs = zc[0:RN]
    var = _fold4(jnp.sum(zcs * zcs, axis=0, keepdims=True), float(N))
    return zc * lax.rsqrt(var + 1e-5) * gam + bet


def _proj_body(x_ref, w1big_ref, o_ref):
    o_ref[0:RN, :] = jnp.dot(x_ref[...], w1big_ref[...],
                             preferred_element_type=jnp.float32)
    o_ref[RN:RP, :] = jnp.zeros((RP - RN, F), jnp.float32)


_proj = pl.pallas_call(_proj_body, out_shape=jax.ShapeDtypeStruct((RP, F), jnp.float32))


def _layer_body(p_ref, parts_ref, b1_ref, w2big_ref, b2_ref, g_ref, be_ref,
                w1nbig_ref, o_ref):
    h = _mlp_bn(p_ref[...], parts_ref[0], parts_ref[1], b1_ref[...],
                w2big_ref[...], b2_ref[...], g_ref[...], be_ref[...])
    o_ref[...] = jnp.dot(h, w1nbig_ref[...], preferred_element_type=jnp.float32)


_layer = pl.pallas_call(_layer_body, out_shape=jax.ShapeDtypeStruct((RP, F), jnp.float32))


def _last_body(p_ref, parts_ref, b1_ref, w2big_ref, b2_ref, g_ref, be_ref, o_ref):
    h = _mlp_bn(p_ref[...], parts_ref[0], parts_ref[1], b1_ref[...],
                w2big_ref[...], b2_ref[...], g_ref[...], be_ref[...])
    o_ref[...] = h
    o_ref[RN:RN + 1, :] = jnp.ones((1, F), jnp.float32)


_last = pl.pallas_call(_last_body, out_shape=jax.ShapeDtypeStruct((RP, F), jnp.float32))


def _head_body(pp_ref, fc1w_ref, fc1b_ref, fc2w_ref, fc2b_ref, o_ref):
    sums = pp_ref[0, 0:G] + pp_ref[1, 0:G]
    counts = pp_ref[0, G:2 * G, 0:1] + pp_ref[1, G:2 * G, 0:1]
    pooled = sums / jnp.maximum(counts, 1.0)
    z = jnp.maximum(jnp.dot(pooled, fc1w_ref[...],
                            preferred_element_type=jnp.float32) + fc1b_ref[...], 0.0)
    logits = jnp.dot(z, fc2w_ref[...], preferred_element_type=jnp.float32) + fc2b_ref[...]
    m = jnp.max(logits, axis=-1, keepdims=True)
    lse = m + jnp.log(jnp.sum(jnp.exp(logits - m), axis=-1, keepdims=True))
    o_ref[...] = logits - lse


_head = pl.pallas_call(_head_body, out_shape=jax.ShapeDtypeStruct((G, CLS), jnp.float32))


def kernel(x, params, edge_index, batch):
    ei = edge_index.astype(jnp.int32)
    bat = batch.astype(jnp.int32)
    pad = EPAD - E
    src2 = jnp.concatenate([ei[0], jnp.zeros((pad,), jnp.int32)]).reshape(NW * K, CH)
    dst2 = jnp.concatenate([ei[1], jnp.full((pad,), N, jnp.int32)]).reshape(NW * K, CH)
    padp = EPOOL - 2 * N
    srcp = jnp.concatenate([jnp.arange(N, dtype=jnp.int32), jnp.full((N,), N, jnp.int32),
                            jnp.zeros((padp,), jnp.int32)]).reshape(NW * KP, CH)
    dstp = jnp.concatenate([bat, bat + G,
                            jnp.full((padp,), NPAD - 1, jnp.int32)]).reshape(NW * KP, CH)
    zeros = jnp.zeros((NPAD, H), jnp.float32)
    x_r = x.reshape(RN, 4 * F)
    eye4 = jnp.eye(4, dtype=jnp.float32)
    big = lambda w: jnp.kron(eye4, w)
    vec4 = lambda v: jnp.tile(v, 4).reshape(1, F)

    segsum = _make_segsum(K)
    p = _proj(x_r, big(params["conv1_W1"]))
    for i in range(1, 6):
        parts = segsum(p.reshape(NPAD, H), src2, dst2, zeros).reshape(NC, RP, F)
        args = (p, parts, vec4(params[f"conv{i}_b1"]), big(params[f"conv{i}_W2"]),
                vec4(params[f"conv{i}_b2"]), vec4(params[f"bn{i}_gamma"]),
                vec4(params[f"bn{i}_beta"]))
        if i < 5:
            p = _layer(*args, big(params[f"conv{i + 1}_W1"]))
        else:
            h5 = _last(*args)
    pool = _make_segsum(KP)(h5.reshape(NPAD, H), srcp, dstp, zeros)
    pp = pool[:, 0:2 * G, :]
    return _head(pp, params["fc1_W"], params["fc1_b"].reshape(1, H),
                 params["fc2_W"], params["fc2_b"].reshape(1, CLS))
